# baseline (speedup 1.0000x reference)
import functools

import jax
import jax.numpy as jnp
from jax import lax
from jax.experimental import pallas as pl
from jax.experimental.pallas import tpu as pltpu
from jax.experimental.pallas import tpu_sc as plsc

N = 10000
E = 320000
D = 128
NG = 256
NW = 32
EPW = E // NW
NCH = 25
CH = 80
NB = 10
BN_ROWS = N // NB
DX = 144

_f32 = jnp.float32


def _sc_edge_body(src_h, dst_h, ea_h, aD_h, c_h, hx_h,
                  acc_out,
                  srcv, dstv, eav, wch, a_d, rows,
                  cv, acc_sp, gsem, ssem, csem):
    ci = lax.axis_index("c")
    si = lax.axis_index("s")
    wid = si * 2 + ci

    pltpu.sync_copy(c_h, cv)

    z16 = jnp.zeros((16,), _f32)

    def zb(i, _):
        for l in range(9):
            rows[i, pl.ds(l * 16, 16)] = z16
        return 0
    lax.fori_loop(0, CH, zb, 0)

    base = si * 624
    for i in range(7):
        pltpu.sync_copy(rows.at[pl.ds(0, CH)],
                        acc_sp.at[pl.ds(base + i * 80, 80)])
    pltpu.sync_copy(rows.at[pl.ds(0, 64)], acc_sp.at[pl.ds(base + 560, 64)])

    @pl.when(si == 15)
    def _():
        pltpu.sync_copy(rows.at[pl.ds(0, 16)], acc_sp.at[pl.ds(9984, 16)])

    plsc.subcore_barrier()

    cvec = cv[pl.ds(0, 16)]
    gb = cvec[0]
    cl = cvec[1]
    iota16 = lax.iota(jnp.int32, 16)
    zi16 = jnp.zeros((16,), jnp.int32)
    dnums = lax.GatherDimensionNumbers(
        offset_dims=(), collapsed_slice_dims=(0,), start_index_map=(0,))

    def _stage(k, pk, sync):
        row = wid * NCH + k
        if sync:
            pltpu.sync_copy(src_h.at[row], srcv.at[pk])
            pltpu.sync_copy(dst_h.at[row], dstv.at[pk])
            pltpu.sync_copy(ea_h.at[row], eav.at[pk])
        else:
            pltpu.async_copy(src_h.at[row], srcv.at[pk], ssem)
            pltpu.async_copy(dst_h.at[row], dstv.at[pk], ssem)
            pltpu.async_copy(ea_h.at[row], eav.at[pk], ssem)

    def _stage_wait(k, pk):
        row = wid * NCH + k
        pltpu.make_async_copy(src_h.at[row], srcv.at[pk], ssem).wait()
        pltpu.make_async_copy(dst_h.at[row], dstv.at[pk], ssem).wait()
        pltpu.make_async_copy(ea_h.at[row], eav.at[pk], ssem).wait()

    def _issue(pk, j, p):
        dsl = pl.ds(p * CH, CH)
        pltpu.async_copy(hx_h.at[srcv.at[pk, j]], rows.at[dsl], gsem)
        pltpu.async_copy(aD_h.at[dstv.at[pk, j]], a_d.at[dsl], gsem)

    def _gather_wait(pk, j, p):
        dsl = pl.ds(p * CH, CH)
        pltpu.make_async_copy(hx_h.at[srcv.at[pk, j]], rows.at[dsl],
                              gsem).wait()
        pltpu.make_async_copy(aD_h.at[dstv.at[pk, j]], a_d.at[dsl],
                              gsem).wait()

    _stage(0, 0, True)
    _issue(0, 0, 0)

    plsc.subcore_barrier()

    def chunk(k, _):
        pk = k & 1

        @pl.when(k < NCH - 1)
        def _():
            _stage(k + 1, 1 - pk, False)

        for j in range(5):
            p = (k + j) & 1
            pb = p * CH
            qb = (1 - p) * CH

            def _scat_wait():
                pltpu.make_async_copy(rows.at[pl.ds(qb, CH)],
                                      acc_sp.at[dstv.at[pk, j]], csem).wait()
            if j == 0:
                @pl.when(k > 0)
                def _():
                    _scat_wait()
            else:
                _scat_wait()

            _gather_wait(pk, j, p)
            if j < 4:
                _issue(pk, j + 1, 1 - p)
            else:
                @pl.when(k < NCH - 1)
                def _():
                    _stage_wait(k + 1, 1 - pk)
                    _issue(1 - pk, 0, 1 - p)
            c129 = jnp.full((16,), 129, jnp.int32)
            for g in range(5):
                sl = pl.ds(g * 16, 16)
                s16 = srcv[pk, j, sl]
                d16 = dstv[pk, j, sl]
                e16 = eav[pk, j, sl]
                g16 = g * 16 + iota16
                a_s16 = plsc.load_gather(rows, [pb + g16, c129])
                a_d16 = plsc.load_gather(a_d, [pb + g16, zi16])
                al = a_s16 + a_d16 + cl * e16
                al = jnp.maximum(al, 0.2 * al)
                b16 = gb + a_d16
                b16 = jnp.maximum(b16, 0.2 * b16)
                w16 = jnp.where(s16 != d16, jnp.exp(al - b16), 0.0)
                wch[sl] = w16

            @plsc.parallel_loop(0, CH, unroll=4)
            def _(r):
                w16 = wch[pl.ds((r // 16) * 16, 16)]
                lane = jnp.full((16, 1), r % 16, jnp.int32)
                wv = lax.gather(w16, lane, dnums, (1,),
                                mode=lax.GatherScatterMode.PROMISE_IN_BOUNDS)
                rr = pb + r
                for l in range(9):
                    sl2 = pl.ds(l * 16, 16)
                    rows[rr, sl2] = rows[rr, sl2] * wv

            pltpu.async_copy(rows.at[pl.ds(pb, CH)],
                             acc_sp.at[dstv.at[pk, j]], csem, add=True)
        return 0
    lax.fori_loop(0, NCH, chunk, 0)

    pltpu.make_async_copy(rows.at[pl.ds(0, CH)],
                          acc_sp.at[dstv.at[0, 0]], csem).wait()

    plsc.subcore_barrier()

    @pl.when(si < 15)
    def _():
        dsl = pl.ds(base, 624)
        pltpu.sync_copy(acc_sp.at[dsl], acc_out.at[ci, dsl])

    @pl.when(si == 15)
    def _():
        dsl = pl.ds(9360, 640)
        pltpu.sync_copy(acc_sp.at[dsl], acc_out.at[ci, dsl])


def _sc_edge(src3, dst3, ea3, aD, consts, hx):
    mesh = plsc.VectorSubcoreMesh(core_axis_name="c", subcore_axis_name="s")
    fn = pl.kernel(
        _sc_edge_body,
        out_type=jax.ShapeDtypeStruct((2, N, DX), _f32),
        name="sc_edge",
        mesh=mesh,
        compiler_params=pltpu.CompilerParams(needs_layout_passes=False,
                                             use_tc_tiling_on_sc=False),
        scratch_types=[
            pltpu.VMEM((2, 5, CH), jnp.int32),
            pltpu.VMEM((2, 5, CH), jnp.int32),
            pltpu.VMEM((2, 5, CH), _f32),
            pltpu.VMEM((CH,), _f32),
            pltpu.VMEM((2 * CH, 16), _f32),
            pltpu.VMEM((2 * CH, DX), _f32),
            pltpu.VMEM((16,), _f32),
            pltpu.VMEM_SHARED((N, DX), _f32),
            pltpu.SemaphoreType.DMA,
            pltpu.SemaphoreType.DMA,
            pltpu.SemaphoreType.DMA,
        ],
    )
    return fn(src3, dst3, ea3, aD, consts, hx)


def _stats_kernel(x_ref, g_ref, b_ref, o_ref, acc):
    i = pl.program_id(0)

    @pl.when(i == 0)
    def _():
        acc[...] = jnp.zeros_like(acc)

    xb = x_ref[...]
    acc[0, :] += jnp.sum(xb, axis=0)
    acc[1, :] += jnp.sum(xb * xb, axis=0)

    @pl.when(i == NB - 1)
    def _():
        mu = acc[0, :] / N
        var = acc[1, :] / N - mu * mu
        scale = g_ref[0, :] * lax.rsqrt(var + 1e-5)
        o_ref[0, :] = scale
        o_ref[1, :] = b_ref[0, :] - mu * scale


def _stats(x, gamma, beta):
    return pl.pallas_call(
        _stats_kernel,
        grid=(NB,),
        in_specs=[pl.BlockSpec((BN_ROWS, D), lambda i: (i, 0)),
                  pl.BlockSpec((1, D), lambda i: (0, 0)),
                  pl.BlockSpec((1, D), lambda i: (0, 0))],
        out_specs=pl.BlockSpec((2, D), lambda i: (0, 0)),
        out_shape=jax.ShapeDtypeStruct((2, D), _f32),
        scratch_shapes=[pltpu.VMEM((2, D), _f32)],
    )(x, gamma.reshape(1, D), beta.reshape(1, D))


def _layer_main_kernel(x_ref, ss_ref, w_ref, as_ref, ad_ref,
                       hx_ref, aS_ref, aD_ref, gm_ref, mx_ref):
    i = pl.program_id(0)
    xb = x_ref[...] * ss_ref[0, :][None, :] + ss_ref[1, :][None, :]
    h = jnp.dot(xb, w_ref[...], preferred_element_type=_f32)
    aSb = jnp.dot(h, as_ref[...], preferred_element_type=_f32)
    aDb = jnp.dot(h, ad_ref[...], preferred_element_type=_f32)
    aS_ref[...] = aSb
    aD_ref[...] = aDb
    col = lax.broadcasted_iota(jnp.int32, (BN_ROWS, 16), 1)
    extra = jnp.where(col == 0, 1.0, jnp.where(col == 1, aSb, 0.0))
    hx_ref[...] = jnp.concatenate([h, extra], axis=1)

    @pl.when(i == 0)
    def _():
        mx_ref[0] = -jnp.inf

    mx_ref[0] = jnp.maximum(mx_ref[0], jnp.max(aSb[:, :1]))

    @pl.when(i == NB - 1)
    def _():
        gm_ref[0] = mx_ref[0]


def _layer_main(x, ss, linN, attS, attD):
    return pl.pallas_call(
        _layer_main_kernel,
        grid=(NB,),
        in_specs=[pl.BlockSpec((BN_ROWS, D), lambda i: (i, 0)),
                  pl.BlockSpec((2, D), lambda i: (0, 0)),
                  pl.BlockSpec((D, D), lambda i: (0, 0)),
                  pl.BlockSpec((D, 16), lambda i: (0, 0)),
                  pl.BlockSpec((D, 16), lambda i: (0, 0))],
        out_specs=[pl.BlockSpec((BN_ROWS, DX), lambda i: (i, 0)),
                   pl.BlockSpec((BN_ROWS, 16), lambda i: (i, 0)),
                   pl.BlockSpec((BN_ROWS, 16), lambda i: (i, 0)),
                   pl.BlockSpec(memory_space=pltpu.SMEM)],
        out_shape=[jax.ShapeDtypeStruct((N, DX), _f32),
                   jax.ShapeDtypeStruct((N, 16), _f32),
                   jax.ShapeDtypeStruct((N, 16), _f32),
                   jax.ShapeDtypeStruct((1,), _f32)],
        scratch_shapes=[pltpu.SMEM((1,), _f32)],
    )(x, ss, linN, attS, attD)


def _minmax_kernel(ea_ref, o_ref, acc_ref):
    i = pl.program_id(0)

    @pl.when(i == 0)
    def _():
        acc_ref[0] = -jnp.inf
        acc_ref[1] = jnp.inf

    eb = ea_ref[...]
    acc_ref[0] = jnp.maximum(acc_ref[0], jnp.max(eb))
    acc_ref[1] = jnp.minimum(acc_ref[1], jnp.min(eb))

    @pl.when(i == NB - 1)
    def _():
        o_ref[0] = acc_ref[0]
        o_ref[1] = acc_ref[1]


def _ea_minmax(ea2d):
    return pl.pallas_call(
        _minmax_kernel,
        grid=(NB,),
        in_specs=[pl.BlockSpec((64, 500), lambda i: (i, 0))],
        out_specs=pl.BlockSpec(memory_space=pltpu.SMEM),
        out_shape=jax.ShapeDtypeStruct((2,), _f32),
        scratch_shapes=[pltpu.SMEM((2,), _f32)],
    )(ea2d)


def _post_kernel(acc_ref, hx_ref, as_ref, ad_ref, gb_ref, bias_ref,
                 o_ref):
    gb = gb_ref[0]
    a = acc_ref[0][:, :D] + acc_ref[1][:, :D]
    sv = (acc_ref[0][:, D:D + 1] + acc_ref[1][:, D:D + 1])
    asf = as_ref[...][:, :1] + ad_ref[...][:, :1]
    asf = jnp.maximum(asf, 0.2 * asf)
    b = gb + ad_ref[...][:, :1]
    b = jnp.maximum(b, 0.2 * b)
    ws = jnp.exp(asf - b)
    hv = hx_ref[...][:, :D]
    out = (a + ws * hv) / (sv + ws) + bias_ref[...] + hv
    o_ref[...] = jnp.maximum(out, 0.0)


def _post(acc2, hx, aS, aD, gb, bias):
    return pl.pallas_call(
        _post_kernel,
        grid=(NB,),
        in_specs=[pl.BlockSpec((2, BN_ROWS, DX), lambda i: (0, i, 0)),
                  pl.BlockSpec((BN_ROWS, DX), lambda i: (i, 0)),
                  pl.BlockSpec((BN_ROWS, 16), lambda i: (i, 0)),
                  pl.BlockSpec((BN_ROWS, 16), lambda i: (i, 0)),
                  pl.BlockSpec(memory_space=pltpu.SMEM),
                  pl.BlockSpec((1, D), lambda i: (0, 0))],
        out_specs=pl.BlockSpec((BN_ROWS, D), lambda i: (i, 0)),
        out_shape=jax.ShapeDtypeStruct((N, D), _f32),
    )(acc2, hx, aS, aD, gb, bias.reshape(1, D))


def _final_kernel(x_ref, fw_ref, fb_ref, bt_ref, w1_ref, b1_ref, w2_ref,
                  b2_ref, wf_ref, bf_ref, o_ref, pool, cnt):
    i = pl.program_id(0)

    @pl.when(i == 0)
    def _():
        pool[...] = jnp.zeros_like(pool)
        cnt[...] = jnp.zeros_like(cnt)

    y = jnp.dot(x_ref[...], fw_ref[...], preferred_element_type=_f32)
    y = y + fb_ref[...]
    oh = (bt_ref[...] == lax.broadcasted_iota(jnp.int32, (BN_ROWS, NG), 1))
    ohf = oh.astype(_f32)
    pool[...] += lax.dot_general(ohf, y, (((0,), (0,)), ((), ())),
                                 preferred_element_type=_f32)
    cnt[...] += lax.dot_general(ohf, jnp.ones((BN_ROWS, 1), _f32),
                                (((0,), (0,)), ((), ())),
                                preferred_element_type=_f32)

    @pl.when(i == NB - 1)
    def _():
        pooled = pool[...] / jnp.maximum(cnt[...], 1.0)
        h1 = jnp.maximum(
            jnp.dot(pooled, w1_ref[...], preferred_element_type=_f32)
            + b1_ref[...], 0.0)
        h2 = jnp.maximum(
            jnp.dot(h1, w2_ref[...], preferred_element_type=_f32)
            + b2_ref[...], 0.0)
        lg = jnp.dot(h2, wf_ref[...], preferred_element_type=_f32) + bf_ref[0]
        m = jnp.max(lg)
        e = jnp.exp(lg - m)
        o_ref[...] = e / jnp.sum(e)


def _final(emb, fW, fb, batchf, W1, b1, W2, b2, Wf, bf):
    return pl.pallas_call(
        _final_kernel,
        grid=(NB,),
        in_specs=[pl.BlockSpec((BN_ROWS, D), lambda i: (i, 0)),
                  pl.BlockSpec((D, D), lambda i: (0, 0)),
                  pl.BlockSpec((1, D), lambda i: (0, 0)),
                  pl.BlockSpec((BN_ROWS, 1), lambda i: (i, 0)),
                  pl.BlockSpec((D, D), lambda i: (0, 0)),
                  pl.BlockSpec((1, D), lambda i: (0, 0)),
                  pl.BlockSpec((D, D), lambda i: (0, 0)),
                  pl.BlockSpec((1, D), lambda i: (0, 0)),
                  pl.BlockSpec((D, 1), lambda i: (0, 0)),
                  pl.BlockSpec(memory_space=pltpu.SMEM)],
        out_specs=pl.BlockSpec((NG, 1), lambda i: (0, 0)),
        out_shape=jax.ShapeDtypeStruct((NG, 1), _f32),
        scratch_shapes=[pltpu.VMEM((NG, D), _f32), pltpu.VMEM((NG, 1), _f32)],
    )(emb, fW, fb.reshape(1, D), batchf, W1, b1.reshape(1, D), W2,
      b2.reshape(1, D), Wf, bf)


def kernel(x, edge_index, edge_attr, batch, params):
    src3 = edge_index[0].astype(jnp.int32).reshape(NW * NCH, 5, CH)
    dst3 = edge_index[1].astype(jnp.int32).reshape(NW * NCH, 5, CH)
    ea3 = edge_attr[:, 0].reshape(NW * NCH, 5, CH)
    batchf = batch.astype(jnp.int32).reshape(N, 1)

    mm = _ea_minmax(edge_attr.reshape(640, 500))

    emb = x
    ones = jnp.ones((D,), _f32)
    zeros = jnp.zeros((D,), _f32)
    for i, p in enumerate(params['layers']):
        if i == 0:
            ss = jnp.stack([ones, zeros])
        else:
            ss = _stats(emb, p['gamma'], p['beta'])
        att = p['att'][0, 0]
        attS = jnp.tile(att[:D].reshape(D, 1), (1, 16))
        attD = jnp.tile(att[D:2 * D].reshape(D, 1), (1, 16))
        cl = p['linE'][0, 0] * att[2 * D]
        hx, aS, aD, gm = _layer_main(emb, ss, p['linN'], attS, attD)
        gmaxc = jnp.maximum(jnp.where(cl > 0, cl * mm[0], cl * mm[1]), 0.0)
        gb = gm[0] + gmaxc
        consts = jnp.concatenate(
            [jnp.stack([gb, cl]), jnp.zeros((14,))]).astype(_f32)
        acc2 = _sc_edge(src3, dst3, ea3, aD, consts, hx)
        emb = _post(acc2, hx, aS, aD, gb.reshape(1), p['bias'])

    out = _final(emb, params['final_W'], params['final_b'], batchf,
                 params['mc_W'][0], params['mc_b'][0],
                 params['mc_W'][1], params['mc_b'][1],
                 params['mc_Wf'], params['mc_bf'])
    return out.reshape(NG)


# speedup vs baseline: 1.1032x; 1.1032x over previous
import functools

import jax
import jax.numpy as jnp
from jax import lax
from jax.experimental import pallas as pl
from jax.experimental.pallas import tpu as pltpu
from jax.experimental.pallas import tpu_sc as plsc

N = 10000
E = 320000
D = 128
NG = 256
NW = 32
EPW = E // NW
NCH = 25
CH = 80
NB = 10
BN_ROWS = N // NB

_f32 = jnp.float32


def _sc_edge_body(src_h, dst_h, ea_h, aS_h, aD_h, c_h, h_h,
                  acc_out, s_out,
                  srcv, dstv, eav, wch, a_s, a_d, rows, srow,
                  cv, acc_sp, sW_sp, gsem, ssem, csem):
    ci = lax.axis_index("c")
    si = lax.axis_index("s")
    wid = si * 2 + ci

    pltpu.sync_copy(c_h, cv)

    z16 = jnp.zeros((16,), _f32)

    def zb(i, _):
        for l in range(8):
            rows[i, pl.ds(l * 16, 16)] = z16
        srow[i] = z16
        srow[CH + i] = z16
        return 0
    lax.fori_loop(0, CH, zb, 0)

    base = si * 624
    for i in range(7):
        pltpu.sync_copy(rows.at[pl.ds(0, CH)],
                        acc_sp.at[pl.ds(base + i * 80, 80)])
        pltpu.sync_copy(srow.at[pl.ds(0, CH)],
                        sW_sp.at[pl.ds(base + i * 80, 80)])
    pltpu.sync_copy(rows.at[pl.ds(0, 64)], acc_sp.at[pl.ds(base + 560, 64)])
    pltpu.sync_copy(srow.at[pl.ds(0, 64)], sW_sp.at[pl.ds(base + 560, 64)])

    @pl.when(si == 15)
    def _():
        pltpu.sync_copy(rows.at[pl.ds(0, 16)], acc_sp.at[pl.ds(9984, 16)])
        pltpu.sync_copy(srow.at[pl.ds(0, 16)], sW_sp.at[pl.ds(9984, 16)])

    plsc.subcore_barrier()

    cvec = cv[pl.ds(0, 16)]
    gb = cvec[0]
    cl = cvec[1]
    iota16 = lax.iota(jnp.int32, 16)
    zi16 = jnp.zeros((16,), jnp.int32)
    dnums = lax.GatherDimensionNumbers(
        offset_dims=(), collapsed_slice_dims=(0,), start_index_map=(0,))

    def _stage(k, pk, sync):
        row = wid * NCH + k
        if sync:
            pltpu.sync_copy(src_h.at[row], srcv.at[pk])
            pltpu.sync_copy(dst_h.at[row], dstv.at[pk])
            pltpu.sync_copy(ea_h.at[row], eav.at[pk])
        else:
            pltpu.async_copy(src_h.at[row], srcv.at[pk], ssem)
            pltpu.async_copy(dst_h.at[row], dstv.at[pk], ssem)
            pltpu.async_copy(ea_h.at[row], eav.at[pk], ssem)

    def _stage_wait(k, pk):
        row = wid * NCH + k
        pltpu.make_async_copy(src_h.at[row], srcv.at[pk], ssem).wait()
        pltpu.make_async_copy(dst_h.at[row], dstv.at[pk], ssem).wait()
        pltpu.make_async_copy(ea_h.at[row], eav.at[pk], ssem).wait()

    def _issue(pk, j, p):
        pb = p * CH
        dsl = pl.ds(pb, CH)
        pltpu.async_copy(h_h.at[srcv.at[pk, j]], rows.at[dsl], gsem)
        pltpu.async_copy(aS_h.at[srcv.at[pk, j]], a_s.at[dsl], gsem)
        pltpu.async_copy(aD_h.at[dstv.at[pk, j]], a_d.at[dsl], gsem)

    def _gather_wait(pk, j, p):
        pb = p * CH
        dsl = pl.ds(pb, CH)
        pltpu.make_async_copy(h_h.at[srcv.at[pk, j]], rows.at[dsl],
                              gsem).wait()
        pltpu.make_async_copy(aS_h.at[srcv.at[pk, j]], a_s.at[dsl],
                              gsem).wait()
        pltpu.make_async_copy(aD_h.at[dstv.at[pk, j]], a_d.at[dsl],
                              gsem).wait()

    _stage(0, 0, True)
    _issue(0, 0, 0)

    plsc.subcore_barrier()

    def chunk(k, _):
        pk = k & 1

        @pl.when(k < NCH - 1)
        def _():
            _stage(k + 1, 1 - pk, False)

        for j in range(5):
            p = (k + j) & 1
            pb = p * CH
            qb = (1 - p) * CH

            def _scat_wait():
                pltpu.make_async_copy(rows.at[pl.ds(qb, CH)],
                                      acc_sp.at[dstv.at[pk, j]], csem).wait()
                pltpu.make_async_copy(srow.at[pl.ds(qb, CH)],
                                      sW_sp.at[dstv.at[pk, j]], csem).wait()
            if j == 0:
                @pl.when(k > 0)
                def _():
                    _scat_wait()
            else:
                _scat_wait()

            _gather_wait(pk, j, p)
            if j < 4:
                _issue(pk, j + 1, 1 - p)
            else:
                @pl.when(k < NCH - 1)
                def _():
                    _stage_wait(k + 1, 1 - pk)
                    _issue(1 - pk, 0, 1 - p)
            for g in range(5):
                sl = pl.ds(g * 16, 16)
                s16 = srcv[pk, j, sl]
                d16 = dstv[pk, j, sl]
                e16 = eav[pk, j, sl]
                g16 = g * 16 + iota16
                a_s16 = plsc.load_gather(a_s, [pb + g16, zi16])
                a_d16 = plsc.load_gather(a_d, [pb + g16, zi16])
                al = a_s16 + a_d16 + cl * e16
                al = jnp.maximum(al, 0.2 * al)
                b16 = gb + a_d16
                b16 = jnp.maximum(b16, 0.2 * b16)
                w16 = jnp.where(s16 != d16, jnp.exp(al - b16), 0.0)
                wch[sl] = w16
                plsc.store_scatter(srow, [pb + g16, zi16], w16)

            @plsc.parallel_loop(0, CH, unroll=4)
            def _(r):
                w16 = wch[pl.ds((r // 16) * 16, 16)]
                lane = jnp.full((16, 1), r % 16, jnp.int32)
                wv = lax.gather(w16, lane, dnums, (1,),
                                mode=lax.GatherScatterMode.PROMISE_IN_BOUNDS)
                rr = pb + r
                for l in range(8):
                    sl2 = pl.ds(l * 16, 16)
                    rows[rr, sl2] = rows[rr, sl2] * wv

            pltpu.async_copy(rows.at[pl.ds(pb, CH)],
                             acc_sp.at[dstv.at[pk, j]], csem, add=True)
            pltpu.async_copy(srow.at[pl.ds(pb, CH)],
                             sW_sp.at[dstv.at[pk, j]], csem, add=True)
        return 0
    lax.fori_loop(0, NCH, chunk, 0)

    pltpu.make_async_copy(rows.at[pl.ds(0, CH)],
                          acc_sp.at[dstv.at[0, 0]], csem).wait()
    pltpu.make_async_copy(srow.at[pl.ds(0, CH)],
                          sW_sp.at[dstv.at[0, 0]], csem).wait()

    plsc.subcore_barrier()

    @pl.when(si < 15)
    def _():
        dsl = pl.ds(base, 624)
        pltpu.sync_copy(acc_sp.at[dsl], acc_out.at[ci, dsl])
        pltpu.sync_copy(sW_sp.at[dsl], s_out.at[ci, dsl])

    @pl.when(si == 15)
    def _():
        dsl = pl.ds(9360, 640)
        pltpu.sync_copy(acc_sp.at[dsl], acc_out.at[ci, dsl])
        pltpu.sync_copy(sW_sp.at[dsl], s_out.at[ci, dsl])


def _sc_edge(src3, dst3, ea3, aS, aD, consts, h):
    mesh = plsc.VectorSubcoreMesh(core_axis_name="c", subcore_axis_name="s")
    fn = pl.kernel(
        _sc_edge_body,
        out_type=[jax.ShapeDtypeStruct((2, N, D), _f32),
                  jax.ShapeDtypeStruct((2, N, 16), _f32)],
        name="sc_edge",
        mesh=mesh,
        compiler_params=pltpu.CompilerParams(needs_layout_passes=False,
                                             use_tc_tiling_on_sc=False),
        scratch_types=[
            pltpu.VMEM((2, 5, CH), jnp.int32),
            pltpu.VMEM((2, 5, CH), jnp.int32),
            pltpu.VMEM((2, 5, CH), _f32),
            pltpu.VMEM((CH,), _f32),
            pltpu.VMEM((2 * CH, 16), _f32),
            pltpu.VMEM((2 * CH, 16), _f32),
            pltpu.VMEM((2 * CH, D), _f32),
            pltpu.VMEM((2 * CH, 16), _f32),
            pltpu.VMEM((16,), _f32),
            pltpu.VMEM_SHARED((N, D), _f32),
            pltpu.VMEM_SHARED((N, 16), _f32),
            pltpu.SemaphoreType.DMA,
            pltpu.SemaphoreType.DMA,
            pltpu.SemaphoreType.DMA,
        ],
    )
    return fn(src3, dst3, ea3, aS, aD, consts, h)


def _stats_kernel(x_ref, g_ref, b_ref, o_ref, acc):
    i = pl.program_id(0)

    @pl.when(i == 0)
    def _():
        acc[...] = jnp.zeros_like(acc)

    xb = x_ref[...]
    acc[0, :] += jnp.sum(xb, axis=0)
    acc[1, :] += jnp.sum(xb * xb, axis=0)

    @pl.when(i == NB - 1)
    def _():
        mu = acc[0, :] / N
        var = acc[1, :] / N - mu * mu
        scale = g_ref[0, :] * lax.rsqrt(var + 1e-5)
        o_ref[0, :] = scale
        o_ref[1, :] = b_ref[0, :] - mu * scale


def _stats(x, gamma, beta):
    return pl.pallas_call(
        _stats_kernel,
        grid=(NB,),
        in_specs=[pl.BlockSpec((BN_ROWS, D), lambda i: (i, 0)),
                  pl.BlockSpec((1, D), lambda i: (0, 0)),
                  pl.BlockSpec((1, D), lambda i: (0, 0))],
        out_specs=pl.BlockSpec((2, D), lambda i: (0, 0)),
        out_shape=jax.ShapeDtypeStruct((2, D), _f32),
        scratch_shapes=[pltpu.VMEM((2, D), _f32)],
    )(x, gamma.reshape(1, D), beta.reshape(1, D))


def _layer_main_kernel(x_ref, ss_ref, w_ref, as_ref, ad_ref,
                       h_ref, aS_ref, aD_ref, gm_ref, mx_ref):
    i = pl.program_id(0)
    xb = x_ref[...] * ss_ref[0, :][None, :] + ss_ref[1, :][None, :]
    h = jnp.dot(xb, w_ref[...], preferred_element_type=_f32)
    h_ref[...] = h
    aSb = jnp.dot(h, as_ref[...], preferred_element_type=_f32)
    aDb = jnp.dot(h, ad_ref[...], preferred_element_type=_f32)
    aS_ref[...] = aSb
    aD_ref[...] = aDb

    @pl.when(i == 0)
    def _():
        mx_ref[0] = -jnp.inf

    mx_ref[0] = jnp.maximum(mx_ref[0], jnp.max(aSb[:, :1]))

    @pl.when(i == NB - 1)
    def _():
        gm_ref[0] = mx_ref[0]


def _layer_main(x, ss, linN, attS, attD):
    return pl.pallas_call(
        _layer_main_kernel,
        grid=(NB,),
        in_specs=[pl.BlockSpec((BN_ROWS, D), lambda i: (i, 0)),
                  pl.BlockSpec((2, D), lambda i: (0, 0)),
                  pl.BlockSpec((D, D), lambda i: (0, 0)),
                  pl.BlockSpec((D, 16), lambda i: (0, 0)),
                  pl.BlockSpec((D, 16), lambda i: (0, 0))],
        out_specs=[pl.BlockSpec((BN_ROWS, D), lambda i: (i, 0)),
                   pl.BlockSpec((BN_ROWS, 16), lambda i: (i, 0)),
                   pl.BlockSpec((BN_ROWS, 16), lambda i: (i, 0)),
                   pl.BlockSpec(memory_space=pltpu.SMEM)],
        out_shape=[jax.ShapeDtypeStruct((N, D), _f32),
                   jax.ShapeDtypeStruct((N, 16), _f32),
                   jax.ShapeDtypeStruct((N, 16), _f32),
                   jax.ShapeDtypeStruct((1,), _f32)],
        scratch_shapes=[pltpu.SMEM((1,), _f32)],
    )(x, ss, linN, attS, attD)


def _minmax_kernel(ea_ref, o_ref, acc_ref):
    i = pl.program_id(0)

    @pl.when(i == 0)
    def _():
        acc_ref[0] = -jnp.inf
        acc_ref[1] = jnp.inf

    eb = ea_ref[...]
    acc_ref[0] = jnp.maximum(acc_ref[0], jnp.max(eb))
    acc_ref[1] = jnp.minimum(acc_ref[1], jnp.min(eb))

    @pl.when(i == NB - 1)
    def _():
        o_ref[0] = acc_ref[0]
        o_ref[1] = acc_ref[1]


def _ea_minmax(ea2d):
    return pl.pallas_call(
        _minmax_kernel,
        grid=(NB,),
        in_specs=[pl.BlockSpec((64, 500), lambda i: (i, 0))],
        out_specs=pl.BlockSpec(memory_space=pltpu.SMEM),
        out_shape=jax.ShapeDtypeStruct((2,), _f32),
        scratch_shapes=[pltpu.SMEM((2,), _f32)],
    )(ea2d)


def _post_block(acc_ref, sw_ref, h_ref, as_ref, ad_ref, gb_ref, bias_ref):
    gb = gb_ref[0]
    a = acc_ref[0] + acc_ref[1]
    sv = sw_ref[0][:, :1] + sw_ref[1][:, :1]
    asf = as_ref[...][:, :1] + ad_ref[...][:, :1]
    asf = jnp.maximum(asf, 0.2 * asf)
    b = gb + ad_ref[...][:, :1]
    b = jnp.maximum(b, 0.2 * b)
    ws = jnp.exp(asf - b)
    hv = h_ref[...]
    out = (a + ws * hv) / (sv + ws) + bias_ref[...] + hv
    return jnp.maximum(out, 0.0)


_POST_SPECS = [pl.BlockSpec((2, BN_ROWS, D), lambda i: (0, i, 0)),
               pl.BlockSpec((2, BN_ROWS, 16), lambda i: (0, i, 0)),
               pl.BlockSpec((BN_ROWS, D), lambda i: (i, 0)),
               pl.BlockSpec((BN_ROWS, 16), lambda i: (i, 0)),
               pl.BlockSpec((BN_ROWS, 16), lambda i: (i, 0)),
               pl.BlockSpec(memory_space=pltpu.SMEM),
               pl.BlockSpec((1, D), lambda i: (0, 0))]


def _post_stats_kernel(acc_ref, sw_ref, h_ref, as_ref, ad_ref, gb_ref,
                       bias_ref, g_ref, b_ref, o_ref, ss_ref, st_ref):
    i = pl.program_id(0)
    out = _post_block(acc_ref, sw_ref, h_ref, as_ref, ad_ref, gb_ref,
                      bias_ref)
    o_ref[...] = out

    @pl.when(i == 0)
    def _():
        st_ref[...] = jnp.zeros_like(st_ref)

    st_ref[0, :] += jnp.sum(out, axis=0)
    st_ref[1, :] += jnp.sum(out * out, axis=0)

    @pl.when(i == NB - 1)
    def _():
        mu = st_ref[0, :] / N
        var = st_ref[1, :] / N - mu * mu
        scale = g_ref[0, :] * lax.rsqrt(var + 1e-5)
        ss_ref[0, :] = scale
        ss_ref[1, :] = b_ref[0, :] - mu * scale


def _post_stats(acc2, s2, h, aS, aD, gb, bias, gamma, beta):
    return pl.pallas_call(
        _post_stats_kernel,
        grid=(NB,),
        in_specs=_POST_SPECS + [pl.BlockSpec((1, D), lambda i: (0, 0)),
                                pl.BlockSpec((1, D), lambda i: (0, 0))],
        out_specs=[pl.BlockSpec((BN_ROWS, D), lambda i: (i, 0)),
                   pl.BlockSpec((2, D), lambda i: (0, 0))],
        out_shape=[jax.ShapeDtypeStruct((N, D), _f32),
                   jax.ShapeDtypeStruct((2, D), _f32)],
        scratch_shapes=[pltpu.VMEM((2, D), _f32)],
    )(acc2, s2, h, aS, aD, gb, bias.reshape(1, D), gamma.reshape(1, D),
      beta.reshape(1, D))


def _final_kernel(acc_ref, sw_ref, h_ref, as_ref, ad_ref, gb_ref, bias_ref,
                  fw_ref, fb_ref, bt_ref, w1_ref, b1_ref, w2_ref,
                  b2_ref, wf_ref, bf_ref, o_ref, pool, cnt):
    i = pl.program_id(0)

    @pl.when(i == 0)
    def _():
        pool[...] = jnp.zeros_like(pool)
        cnt[...] = jnp.zeros_like(cnt)

    emb = _post_block(acc_ref, sw_ref, h_ref, as_ref, ad_ref, gb_ref,
                      bias_ref)
    y = jnp.dot(emb, fw_ref[...], preferred_element_type=_f32)
    y = y + fb_ref[...]
    oh = (bt_ref[...] == lax.broadcasted_iota(jnp.int32, (BN_ROWS, NG), 1))
    ohf = oh.astype(_f32)
    pool[...] += lax.dot_general(ohf, y, (((0,), (0,)), ((), ())),
                                 preferred_element_type=_f32)
    cnt[...] += lax.dot_general(ohf, jnp.ones((BN_ROWS, 1), _f32),
                                (((0,), (0,)), ((), ())),
                                preferred_element_type=_f32)

    @pl.when(i == NB - 1)
    def _():
        pooled = pool[...] / jnp.maximum(cnt[...], 1.0)
        h1 = jnp.maximum(
            jnp.dot(pooled, w1_ref[...], preferred_element_type=_f32)
            + b1_ref[...], 0.0)
        h2 = jnp.maximum(
            jnp.dot(h1, w2_ref[...], preferred_element_type=_f32)
            + b2_ref[...], 0.0)
        lg = jnp.dot(h2, wf_ref[...], preferred_element_type=_f32) + bf_ref[0]
        m = jnp.max(lg)
        e = jnp.exp(lg - m)
        o_ref[...] = e / jnp.sum(e)


def _final(acc2, s2, h, aS, aD, gb, bias, fW, fb, batchf,
           W1, b1, W2, b2, Wf, bf):
    return pl.pallas_call(
        _final_kernel,
        grid=(NB,),
        in_specs=_POST_SPECS + [
                  pl.BlockSpec((D, D), lambda i: (0, 0)),
                  pl.BlockSpec((1, D), lambda i: (0, 0)),
                  pl.BlockSpec((BN_ROWS, 1), lambda i: (i, 0)),
                  pl.BlockSpec((D, D), lambda i: (0, 0)),
                  pl.BlockSpec((1, D), lambda i: (0, 0)),
                  pl.BlockSpec((D, D), lambda i: (0, 0)),
                  pl.BlockSpec((1, D), lambda i: (0, 0)),
                  pl.BlockSpec((D, 1), lambda i: (0, 0)),
                  pl.BlockSpec(memory_space=pltpu.SMEM)],
        out_specs=pl.BlockSpec((NG, 1), lambda i: (0, 0)),
        out_shape=jax.ShapeDtypeStruct((NG, 1), _f32),
        scratch_shapes=[pltpu.VMEM((NG, D), _f32), pltpu.VMEM((NG, 1), _f32)],
    )(acc2, s2, h, aS, aD, gb, bias.reshape(1, D),
      fW, fb.reshape(1, D), batchf, W1, b1.reshape(1, D), W2,
      b2.reshape(1, D), Wf, bf)


def kernel(x, edge_index, edge_attr, batch, params):
    src3 = edge_index[0].astype(jnp.int32).reshape(NW * NCH, 5, CH)
    dst3 = edge_index[1].astype(jnp.int32).reshape(NW * NCH, 5, CH)
    ea3 = edge_attr[:, 0].reshape(NW * NCH, 5, CH)
    batchf = batch.astype(jnp.int32).reshape(N, 1)

    mm = _ea_minmax(edge_attr.reshape(640, 500))

    emb = x
    ones = jnp.ones((D,), _f32)
    zeros = jnp.zeros((D,), _f32)
    ss = jnp.stack([ones, zeros])
    layers = params['layers']
    for i, p in enumerate(layers):
        att = p['att'][0, 0]
        attS = jnp.tile(att[:D].reshape(D, 1), (1, 16))
        attD = jnp.tile(att[D:2 * D].reshape(D, 1), (1, 16))
        cl = p['linE'][0, 0] * att[2 * D]
        h, aS, aD, gm = _layer_main(emb, ss, p['linN'], attS, attD)
        gmaxc = jnp.maximum(jnp.where(cl > 0, cl * mm[0], cl * mm[1]), 0.0)
        gb = gm[0] + gmaxc
        consts = jnp.concatenate(
            [jnp.stack([gb, cl]), jnp.zeros((14,))]).astype(_f32)
        acc2, s2 = _sc_edge(src3, dst3, ea3, aS, aD, consts, h)
        if i < len(layers) - 1:
            emb, ss = _post_stats(acc2, s2, h, aS, aD, gb.reshape(1),
                                  p['bias'], layers[i + 1]['gamma'],
                                  layers[i + 1]['beta'])

    out = _final(acc2, s2, h, aS, aD, gb.reshape(1), p['bias'],
                 params['final_W'], params['final_b'], batchf,
                 params['mc_W'][0], params['mc_b'][0],
                 params['mc_W'][1], params['mc_b'][1],
                 params['mc_Wf'], params['mc_bf'])
    return out.reshape(NG)


# speedup vs baseline: 1.1736x; 1.0638x over previous
import functools

import jax
import jax.numpy as jnp
from jax import lax
from jax.experimental import pallas as pl
from jax.experimental.pallas import tpu as pltpu
from jax.experimental.pallas import tpu_sc as plsc

N = 10000
E = 320000
D = 128
NG = 256
NW = 32
EPW = E // NW
NCH = 25
CH = 80
NB = 10
BN_ROWS = N // NB

_f32 = jnp.float32


def _sc_edge_body(src_h, dst_h, ea_h, aS_h, aD_h, c_h, h_h,
                  acc_out, s_out,
                  srcv, dstv, eav, wch, aSv, aDv, rows, srow,
                  cv, acc_sp, sW_sp, gsem, ssem, csem):
    ci = lax.axis_index("c")
    si = lax.axis_index("s")
    wid = si * 2 + ci

    pltpu.sync_copy(c_h, cv)
    pltpu.sync_copy(aS_h, aSv)
    pltpu.sync_copy(aD_h, aDv)

    z16 = jnp.zeros((16,), _f32)

    iota16z = lax.iota(jnp.int32, 16)

    def zb(i, _):
        for l in range(8):
            rows[i, pl.ds(l * 16, 16)] = z16
        plsc.store_scatter(srow, [i * 2 + (iota16z // 8), iota16z % 8], z16)
        return 0
    lax.fori_loop(0, CH, zb, 0)

    base = si * 624
    for i in range(7):
        pltpu.sync_copy(rows.at[pl.ds(0, CH)],
                        acc_sp.at[pl.ds(base + i * 80, 80)])
        pltpu.sync_copy(srow.at[pl.ds(0, CH)],
                        sW_sp.at[pl.ds(base + i * 80, 80)])
    pltpu.sync_copy(rows.at[pl.ds(0, 64)], acc_sp.at[pl.ds(base + 560, 64)])
    pltpu.sync_copy(srow.at[pl.ds(0, 64)], sW_sp.at[pl.ds(base + 560, 64)])

    @pl.when(si == 15)
    def _():
        pltpu.sync_copy(rows.at[pl.ds(0, 16)], acc_sp.at[pl.ds(9984, 16)])
        pltpu.sync_copy(srow.at[pl.ds(0, 16)], sW_sp.at[pl.ds(9984, 16)])

    plsc.subcore_barrier()

    cvec = cv[pl.ds(0, 16)]
    gb = cvec[0]
    cl = cvec[1]
    iota16 = lax.iota(jnp.int32, 16)
    zi16 = jnp.zeros((16,), jnp.int32)
    dnums = lax.GatherDimensionNumbers(
        offset_dims=(), collapsed_slice_dims=(0,), start_index_map=(0,))

    def _stage(k, pk, sync):
        row = wid * NCH + k
        if sync:
            pltpu.sync_copy(src_h.at[row], srcv.at[pk])
            pltpu.sync_copy(dst_h.at[row], dstv.at[pk])
            pltpu.sync_copy(ea_h.at[row], eav.at[pk])
        else:
            pltpu.async_copy(src_h.at[row], srcv.at[pk], ssem)
            pltpu.async_copy(dst_h.at[row], dstv.at[pk], ssem)
            pltpu.async_copy(ea_h.at[row], eav.at[pk], ssem)

    def _stage_wait(k, pk):
        row = wid * NCH + k
        pltpu.make_async_copy(src_h.at[row], srcv.at[pk], ssem).wait()
        pltpu.make_async_copy(dst_h.at[row], dstv.at[pk], ssem).wait()
        pltpu.make_async_copy(ea_h.at[row], eav.at[pk], ssem).wait()

    def _issue(pk, j, p):
        pb = p * CH
        dsl = pl.ds(pb, CH)
        pltpu.async_copy(h_h.at[srcv.at[pk, j]], rows.at[dsl], gsem)

    def _gather_wait(pk, j, p):
        pb = p * CH
        dsl = pl.ds(pb, CH)
        pltpu.make_async_copy(h_h.at[srcv.at[pk, j]], rows.at[dsl],
                              gsem).wait()

    _stage(0, 0, True)
    _issue(0, 0, 0)

    plsc.subcore_barrier()

    def chunk(k, _):
        pk = k & 1

        @pl.when(k < NCH - 1)
        def _():
            _stage(k + 1, 1 - pk, False)

        for j in range(5):
            p = (k + j) & 1
            pb = p * CH
            qb = (1 - p) * CH

            def _scat_wait():
                pltpu.make_async_copy(rows.at[pl.ds(qb, CH)],
                                      acc_sp.at[dstv.at[pk, j]], csem).wait()
                pltpu.make_async_copy(srow.at[pl.ds(qb, CH)],
                                      sW_sp.at[dstv.at[pk, j]], csem).wait()
            if j == 0:
                @pl.when(k > 0)
                def _():
                    _scat_wait()
            else:
                _scat_wait()

            _gather_wait(pk, j, p)
            if j < 4:
                _issue(pk, j + 1, 1 - p)
            else:
                @pl.when(k < NCH - 1)
                def _():
                    _stage_wait(k + 1, 1 - pk)
                    _issue(1 - pk, 0, 1 - p)
            for g in range(5):
                sl = pl.ds(g * 16, 16)
                s16 = srcv[pk, j, sl]
                d16 = dstv[pk, j, sl]
                e16 = eav[pk, j, sl]
                g16 = g * 16 + iota16
                a_s16 = plsc.load_gather(aSv, [s16])
                a_d16 = plsc.load_gather(aDv, [d16])
                al = a_s16 + a_d16 + cl * e16
                al = jnp.maximum(al, 0.2 * al)
                b16 = gb + a_d16
                b16 = jnp.maximum(b16, 0.2 * b16)
                w16 = jnp.where(s16 != d16, jnp.exp(al - b16), 0.0)
                wch[sl] = w16
                plsc.store_scatter(srow, [pb + g16, zi16], w16)

            @plsc.parallel_loop(0, CH, unroll=4)
            def _(r):
                w16 = wch[pl.ds((r // 16) * 16, 16)]
                lane = jnp.full((16, 1), r % 16, jnp.int32)
                wv = lax.gather(w16, lane, dnums, (1,),
                                mode=lax.GatherScatterMode.PROMISE_IN_BOUNDS)
                rr = pb + r
                for l in range(8):
                    sl2 = pl.ds(l * 16, 16)
                    rows[rr, sl2] = rows[rr, sl2] * wv

            pltpu.async_copy(rows.at[pl.ds(pb, CH)],
                             acc_sp.at[dstv.at[pk, j]], csem, add=True)
            pltpu.async_copy(srow.at[pl.ds(pb, CH)],
                             sW_sp.at[dstv.at[pk, j]], csem, add=True)
        return 0
    lax.fori_loop(0, NCH, chunk, 0)

    pltpu.make_async_copy(rows.at[pl.ds(0, CH)],
                          acc_sp.at[dstv.at[0, 0]], csem).wait()
    pltpu.make_async_copy(srow.at[pl.ds(0, CH)],
                          sW_sp.at[dstv.at[0, 0]], csem).wait()

    plsc.subcore_barrier()

    @pl.when(si < 15)
    def _():
        dsl = pl.ds(base, 624)
        pltpu.sync_copy(acc_sp.at[dsl], acc_out.at[ci, dsl])
        pltpu.sync_copy(sW_sp.at[dsl], s_out.at[ci, dsl])

    @pl.when(si == 15)
    def _():
        dsl = pl.ds(9360, 640)
        pltpu.sync_copy(acc_sp.at[dsl], acc_out.at[ci, dsl])
        pltpu.sync_copy(sW_sp.at[dsl], s_out.at[ci, dsl])


def _sc_edge(src3, dst3, ea3, aS, aD, consts, h):
    mesh = plsc.VectorSubcoreMesh(core_axis_name="c", subcore_axis_name="s")
    fn = pl.kernel(
        _sc_edge_body,
        out_type=[jax.ShapeDtypeStruct((2, N, D), _f32),
                  jax.ShapeDtypeStruct((2, N, 8), _f32)],
        name="sc_edge",
        mesh=mesh,
        compiler_params=pltpu.CompilerParams(needs_layout_passes=False,
                                             use_tc_tiling_on_sc=False),
        scratch_types=[
            pltpu.VMEM((2, 5, CH), jnp.int32),
            pltpu.VMEM((2, 5, CH), jnp.int32),
            pltpu.VMEM((2, 5, CH), _f32),
            pltpu.VMEM((CH,), _f32),
            pltpu.VMEM((N,), _f32),
            pltpu.VMEM((N,), _f32),
            pltpu.VMEM((2 * CH, D), _f32),
            pltpu.VMEM((2 * CH, 8), _f32),
            pltpu.VMEM((16,), _f32),
            pltpu.VMEM_SHARED((N, D), _f32),
            pltpu.VMEM_SHARED((N, 8), _f32),
            pltpu.SemaphoreType.DMA,
            pltpu.SemaphoreType.DMA,
            pltpu.SemaphoreType.DMA,
        ],
    )
    return fn(src3, dst3, ea3, aS, aD, consts, h)


def _stats_kernel(x_ref, g_ref, b_ref, o_ref, acc):
    i = pl.program_id(0)

    @pl.when(i == 0)
    def _():
        acc[...] = jnp.zeros_like(acc)

    xb = x_ref[...]
    acc[0, :] += jnp.sum(xb, axis=0)
    acc[1, :] += jnp.sum(xb * xb, axis=0)

    @pl.when(i == NB - 1)
    def _():
        mu = acc[0, :] / N
        var = acc[1, :] / N - mu * mu
        scale = g_ref[0, :] * lax.rsqrt(var + 1e-5)
        o_ref[0, :] = scale
        o_ref[1, :] = b_ref[0, :] - mu * scale


def _stats(x, gamma, beta):
    return pl.pallas_call(
        _stats_kernel,
        grid=(NB,),
        in_specs=[pl.BlockSpec((BN_ROWS, D), lambda i: (i, 0)),
                  pl.BlockSpec((1, D), lambda i: (0, 0)),
                  pl.BlockSpec((1, D), lambda i: (0, 0))],
        out_specs=pl.BlockSpec((2, D), lambda i: (0, 0)),
        out_shape=jax.ShapeDtypeStruct((2, D), _f32),
        scratch_shapes=[pltpu.VMEM((2, D), _f32)],
    )(x, gamma.reshape(1, D), beta.reshape(1, D))


def _layer_main_kernel(x_ref, ss_ref, w_ref, as_ref, ad_ref,
                       h_ref, aS_ref, aD_ref, gm_ref, mx_ref):
    i = pl.program_id(0)
    xb = x_ref[...] * ss_ref[0, :][None, :] + ss_ref[1, :][None, :]
    h = jnp.dot(xb, w_ref[...], preferred_element_type=_f32)
    h_ref[...] = h
    aSb = jnp.dot(h, as_ref[...], preferred_element_type=_f32)
    aDb = jnp.dot(h, ad_ref[...], preferred_element_type=_f32)
    aS_ref[...] = aSb
    aD_ref[...] = aDb

    @pl.when(i == 0)
    def _():
        mx_ref[0] = -jnp.inf

    mx_ref[0] = jnp.maximum(mx_ref[0], jnp.max(aSb))

    @pl.when(i == NB - 1)
    def _():
        gm_ref[0] = mx_ref[0]


def _layer_main(x, ss, linN, attS, attD):
    return pl.pallas_call(
        _layer_main_kernel,
        grid=(NB,),
        in_specs=[pl.BlockSpec((BN_ROWS, D), lambda i: (i, 0)),
                  pl.BlockSpec((2, D), lambda i: (0, 0)),
                  pl.BlockSpec((D, D), lambda i: (0, 0)),
                  pl.BlockSpec((D, 1), lambda i: (0, 0)),
                  pl.BlockSpec((D, 1), lambda i: (0, 0))],
        out_specs=[pl.BlockSpec((BN_ROWS, D), lambda i: (i, 0)),
                   pl.BlockSpec((BN_ROWS, 1), lambda i: (i, 0)),
                   pl.BlockSpec((BN_ROWS, 1), lambda i: (i, 0)),
                   pl.BlockSpec(memory_space=pltpu.SMEM)],
        out_shape=[jax.ShapeDtypeStruct((N, D), _f32),
                   jax.ShapeDtypeStruct((N, 1), _f32),
                   jax.ShapeDtypeStruct((N, 1), _f32),
                   jax.ShapeDtypeStruct((1,), _f32)],
        scratch_shapes=[pltpu.SMEM((1,), _f32)],
    )(x, ss, linN, attS, attD)


def _minmax_kernel(ea_ref, o_ref, acc_ref):
    i = pl.program_id(0)

    @pl.when(i == 0)
    def _():
        acc_ref[0] = -jnp.inf
        acc_ref[1] = jnp.inf

    eb = ea_ref[...]
    acc_ref[0] = jnp.maximum(acc_ref[0], jnp.max(eb))
    acc_ref[1] = jnp.minimum(acc_ref[1], jnp.min(eb))

    @pl.when(i == NB - 1)
    def _():
        o_ref[0] = acc_ref[0]
        o_ref[1] = acc_ref[1]


def _ea_minmax(ea2d):
    return pl.pallas_call(
        _minmax_kernel,
        grid=(NB,),
        in_specs=[pl.BlockSpec((64, 500), lambda i: (i, 0))],
        out_specs=pl.BlockSpec(memory_space=pltpu.SMEM),
        out_shape=jax.ShapeDtypeStruct((2,), _f32),
        scratch_shapes=[pltpu.SMEM((2,), _f32)],
    )(ea2d)


def _post_block(acc_ref, sw_ref, h_ref, as_ref, ad_ref, gb_ref, bias_ref):
    gb = gb_ref[0]
    a = acc_ref[0] + acc_ref[1]
    sv = sw_ref[0][:, :1] + sw_ref[1][:, :1]
    asf = as_ref[...][:, :1] + ad_ref[...][:, :1]
    asf = jnp.maximum(asf, 0.2 * asf)
    b = gb + ad_ref[...][:, :1]
    b = jnp.maximum(b, 0.2 * b)
    ws = jnp.exp(asf - b)
    hv = h_ref[...]
    out = (a + ws * hv) / (sv + ws) + bias_ref[...] + hv
    return jnp.maximum(out, 0.0)


_POST_SPECS = [pl.BlockSpec((2, BN_ROWS, D), lambda i: (0, i, 0)),
               pl.BlockSpec((2, BN_ROWS, 8), lambda i: (0, i, 0)),
               pl.BlockSpec((BN_ROWS, D), lambda i: (i, 0)),
               pl.BlockSpec((BN_ROWS, 1), lambda i: (i, 0)),
               pl.BlockSpec((BN_ROWS, 1), lambda i: (i, 0)),
               pl.BlockSpec(memory_space=pltpu.SMEM),
               pl.BlockSpec((1, D), lambda i: (0, 0))]


def _post_stats_kernel(acc_ref, sw_ref, h_ref, as_ref, ad_ref, gb_ref,
                       bias_ref, g_ref, b_ref, o_ref, ss_ref, st_ref):
    i = pl.program_id(0)
    out = _post_block(acc_ref, sw_ref, h_ref, as_ref, ad_ref, gb_ref,
                      bias_ref)
    o_ref[...] = out

    @pl.when(i == 0)
    def _():
        st_ref[...] = jnp.zeros_like(st_ref)

    st_ref[0, :] += jnp.sum(out, axis=0)
    st_ref[1, :] += jnp.sum(out * out, axis=0)

    @pl.when(i == NB - 1)
    def _():
        mu = st_ref[0, :] / N
        var = st_ref[1, :] / N - mu * mu
        scale = g_ref[0, :] * lax.rsqrt(var + 1e-5)
        ss_ref[0, :] = scale
        ss_ref[1, :] = b_ref[0, :] - mu * scale


def _post_stats(acc2, s2, h, aS, aD, gb, bias, gamma, beta):
    return pl.pallas_call(
        _post_stats_kernel,
        grid=(NB,),
        in_specs=_POST_SPECS + [pl.BlockSpec((1, D), lambda i: (0, 0)),
                                pl.BlockSpec((1, D), lambda i: (0, 0))],
        out_specs=[pl.BlockSpec((BN_ROWS, D), lambda i: (i, 0)),
                   pl.BlockSpec((2, D), lambda i: (0, 0))],
        out_shape=[jax.ShapeDtypeStruct((N, D), _f32),
                   jax.ShapeDtypeStruct((2, D), _f32)],
        scratch_shapes=[pltpu.VMEM((2, D), _f32)],
    )(acc2, s2, h, aS, aD, gb, bias.reshape(1, D), gamma.reshape(1, D),
      beta.reshape(1, D))


def _final_kernel(acc_ref, sw_ref, h_ref, as_ref, ad_ref, gb_ref, bias_ref,
                  fw_ref, fb_ref, bt_ref, w1_ref, b1_ref, w2_ref,
                  b2_ref, wf_ref, bf_ref, o_ref, pool, cnt):
    i = pl.program_id(0)

    @pl.when(i == 0)
    def _():
        pool[...] = jnp.zeros_like(pool)
        cnt[...] = jnp.zeros_like(cnt)

    emb = _post_block(acc_ref, sw_ref, h_ref, as_ref, ad_ref, gb_ref,
                      bias_ref)
    y = jnp.dot(emb, fw_ref[...], preferred_element_type=_f32)
    y = y + fb_ref[...]
    oh = (bt_ref[...] == lax.broadcasted_iota(jnp.int32, (BN_ROWS, NG), 1))
    ohf = oh.astype(_f32)
    pool[...] += lax.dot_general(ohf, y, (((0,), (0,)), ((), ())),
                                 preferred_element_type=_f32)
    cnt[...] += lax.dot_general(ohf, jnp.ones((BN_ROWS, 1), _f32),
                                (((0,), (0,)), ((), ())),
                                preferred_element_type=_f32)

    @pl.when(i == NB - 1)
    def _():
        pooled = pool[...] / jnp.maximum(cnt[...], 1.0)
        h1 = jnp.maximum(
            jnp.dot(pooled, w1_ref[...], preferred_element_type=_f32)
            + b1_ref[...], 0.0)
        h2 = jnp.maximum(
            jnp.dot(h1, w2_ref[...], preferred_element_type=_f32)
            + b2_ref[...], 0.0)
        lg = jnp.dot(h2, wf_ref[...], preferred_element_type=_f32) + bf_ref[0]
        m = jnp.max(lg)
        e = jnp.exp(lg - m)
        o_ref[...] = e / jnp.sum(e)


def _final(acc2, s2, h, aS, aD, gb, bias, fW, fb, batchf,
           W1, b1, W2, b2, Wf, bf):
    return pl.pallas_call(
        _final_kernel,
        grid=(NB,),
        in_specs=_POST_SPECS + [
                  pl.BlockSpec((D, D), lambda i: (0, 0)),
                  pl.BlockSpec((1, D), lambda i: (0, 0)),
                  pl.BlockSpec((BN_ROWS, 1), lambda i: (i, 0)),
                  pl.BlockSpec((D, D), lambda i: (0, 0)),
                  pl.BlockSpec((1, D), lambda i: (0, 0)),
                  pl.BlockSpec((D, D), lambda i: (0, 0)),
                  pl.BlockSpec((1, D), lambda i: (0, 0)),
                  pl.BlockSpec((D, 1), lambda i: (0, 0)),
                  pl.BlockSpec(memory_space=pltpu.SMEM)],
        out_specs=pl.BlockSpec((NG, 1), lambda i: (0, 0)),
        out_shape=jax.ShapeDtypeStruct((NG, 1), _f32),
        scratch_shapes=[pltpu.VMEM((NG, D), _f32), pltpu.VMEM((NG, 1), _f32)],
    )(acc2, s2, h, aS, aD, gb, bias.reshape(1, D),
      fW, fb.reshape(1, D), batchf, W1, b1.reshape(1, D), W2,
      b2.reshape(1, D), Wf, bf)


def kernel(x, edge_index, edge_attr, batch, params):
    src3 = edge_index[0].astype(jnp.int32).reshape(NW * NCH, 5, CH)
    dst3 = edge_index[1].astype(jnp.int32).reshape(NW * NCH, 5, CH)
    ea3 = edge_attr[:, 0].reshape(NW * NCH, 5, CH)
    batchf = batch.astype(jnp.int32).reshape(N, 1)

    mm = _ea_minmax(edge_attr.reshape(640, 500))

    emb = x
    ones = jnp.ones((D,), _f32)
    zeros = jnp.zeros((D,), _f32)
    ss = jnp.stack([ones, zeros])
    layers = params['layers']
    for i, p in enumerate(layers):
        att = p['att'][0, 0]
        attS = att[:D].reshape(D, 1)
        attD = att[D:2 * D].reshape(D, 1)
        cl = p['linE'][0, 0] * att[2 * D]
        h, aS, aD, gm = _layer_main(emb, ss, p['linN'], attS, attD)
        gmaxc = jnp.maximum(jnp.where(cl > 0, cl * mm[0], cl * mm[1]), 0.0)
        gb = gm[0] + gmaxc
        consts = jnp.concatenate(
            [jnp.stack([gb, cl]), jnp.zeros((14,))]).astype(_f32)
        acc2, s2 = _sc_edge(src3, dst3, ea3, aS.reshape(N), aD.reshape(N),
                            consts, h)
        if i < len(layers) - 1:
            emb, ss = _post_stats(acc2, s2, h, aS, aD, gb.reshape(1),
                                  p['bias'], layers[i + 1]['gamma'],
                                  layers[i + 1]['beta'])

    out = _final(acc2, s2, h, aS, aD, gb.reshape(1), p['bias'],
                 params['final_W'], params['final_b'], batchf,
                 params['mc_W'][0], params['mc_b'][0],
                 params['mc_W'][1], params['mc_b'][1],
                 params['mc_Wf'], params['mc_bf'])
    return out.reshape(NG)
